# baseline (speedup 1.0000x reference)
import functools

import jax
import jax.numpy as jnp
from jax import lax
from jax.experimental import pallas as pl
from jax.experimental.pallas import tpu as pltpu
from jax.experimental.pallas import tpu_sc as plsc

N_NODES = 10000
N_EDGES = 320000
N_TILES = 32
EPT = N_EDGES // N_TILES
CHUNK = 80
RPT = N_NODES // 16


def _prep1_body(x_ref, w1_ref, al_ref, ar_ref, t1_ref, er1_ref):
    feat = jnp.dot(x_ref[...], w1_ref[...], preferred_element_type=jnp.float32)
    r = lax.broadcasted_iota(jnp.int32, (128, 8), 0) // 16
    c = lax.broadcasted_iota(jnp.int32, (128, 8), 1)
    sel = (r == c).astype(jnp.float32)
    el = jnp.dot(feat * al_ref[...], sel, preferred_element_type=jnp.float32)
    er = jnp.dot(feat * ar_ref[...], sel, preferred_element_type=jnp.float32)
    z8 = jnp.zeros_like(el)
    t1_ref[...] = jnp.concatenate([el, z8, feat], axis=1)
    er1_ref[...] = jnp.concatenate([er, z8], axis=1)


def _mid_body(acc_ref, x_ref, b1_ref, w2_ref, al2_ref, ar2_ref,
              h_ref, t2_ref, er2_ref):
    rows = acc_ref[0] + acc_ref[1]
    denom = rows[:, 0:8]
    msg = rows[:, 16:144]
    r = lax.broadcasted_iota(jnp.int32, (8, 128), 1) // 16
    c = lax.broadcasted_iota(jnp.int32, (8, 128), 0)
    sel = (r == c).astype(jnp.float32)
    den_e = jnp.maximum(
        jnp.dot(denom, sel, preferred_element_type=jnp.float32), 1e-9)
    rst = msg / den_e + x_ref[...] + b1_ref[...]
    h = jnp.where(rst > 0, rst, jnp.exp(rst) - 1.0)
    h_ref[...] = h
    feat2 = jnp.dot(h, w2_ref[...], preferred_element_type=jnp.float32)
    el2 = jnp.sum(feat2 * al2_ref[...], axis=1, keepdims=True)
    er2 = jnp.sum(feat2 * ar2_ref[...], axis=1, keepdims=True)
    z15 = jnp.zeros((el2.shape[0], 15), jnp.float32)
    t2_ref[...] = jnp.concatenate([el2, z15, feat2], axis=1)
    er2_ref[...] = jnp.concatenate([er2, z15], axis=1)


def _fin_body(acc_ref, h_ref, rw_ref, b2_ref, out_ref):
    rows = acc_ref[0] + acc_ref[1]
    denom = jnp.maximum(rows[:, 0:1], 1e-9)
    msg = rows[:, 16:80]
    res = jnp.dot(h_ref[...], rw_ref[...], preferred_element_type=jnp.float32)
    out_ref[...] = msg / denom + res + b2_ref[...]


def _edge_pass(table, er_table, eidx, zrows, *, row, nh):
    feat_regs = (row - 16) // 16
    nch = EPT // CHUNK
    mesh = plsc.VectorSubcoreMesh(core_axis_name="c", subcore_axis_name="s")

    def body(t_hbm, er_hbm, eidx_hbm, z_hbm, out_hbm,
             idx0, idx1, idx2, idx3, tr_a, tr_b, er_a, er_b, ob, acc,
             s_ta, s_tb, s_ea, s_eb, s_i0, s_i1, s_i2, s_i3, s_s):
        ci = lax.axis_index("c")
        si = lax.axis_index("s")
        tile = ci * 16 + si
        zbase = si * RPT
        rb = tile * nch
        pltpu.sync_copy(z_hbm, acc.at[pl.ds(zbase, RPT)])
        plsc.subcore_barrier()

        idxs = ((idx0, s_i0), (idx1, s_i1), (idx2, s_i2), (idx3, s_i3))
        trs = ((tr_a, s_ta, er_a, s_ea), (tr_b, s_tb, er_b, s_eb))

        def start_gather(j):
            tr, s_t, err, s_e = trs[j % 2]
            idx = idxs[j % 4][0]
            pltpu.async_copy(t_hbm.at[idx.at[0]], tr, s_t)
            pltpu.async_copy(er_hbm.at[idx.at[1]], err, s_e)

        def compute(tr, err):
            @plsc.parallel_loop(0, CHUNK, unroll=8)
            def edge(i):
                el = tr[i, pl.ds(0, 16)]
                er = err[i, pl.ds(0, 16)]
                e = el + er
                e = jnp.maximum(e, 0.2 * e)
                ex = jnp.exp(e)
                ob[i, pl.ds(0, 16)] = ex
                for j in range(feat_regs):
                    hh = j if nh > 1 else 0
                    a = ex.at[jnp.full((16,), hh, jnp.int32)].get(
                        mode="promise_in_bounds")
                    f = tr[i, pl.ds(16 + 16 * j, 16)]
                    ob[i, pl.ds(16 + 16 * j, 16)] = f * a

        def halfstep(k, j):
            tr, s_t, err, s_e = trs[j % 2]
            idx = idxs[j % 4][0]

            @pl.when(k + 1 < nch)
            def _():
                nidx, n_si = idxs[(j + 1) % 4]
                pltpu.make_async_copy(eidx_hbm.at[0], nidx, n_si).wait()
                start_gather(j + 1)

            @pl.when(k + 2 < nch)
            def _():
                nidx2, n_si2 = idxs[(j + 2) % 4]
                pltpu.async_copy(eidx_hbm.at[rb + k + 2], nidx2, n_si2)

            pltpu.make_async_copy(t_hbm.at[idx.at[0]], tr, s_t).wait()
            pltpu.make_async_copy(er_hbm.at[idx.at[1]], err, s_e).wait()

            @pl.when(k >= 1)
            def _():
                pltpu.make_async_copy(ob, acc.at[idx.at[1]], s_s).wait()

            compute(tr, err)
            pltpu.async_copy(ob, acc.at[idx.at[1]], s_s, add=True)

        pltpu.async_copy(eidx_hbm.at[rb], idx0, s_i0)
        pltpu.async_copy(eidx_hbm.at[rb + 1], idx1, s_i1)
        pltpu.make_async_copy(eidx_hbm.at[0], idx0, s_i0).wait()
        start_gather(0)

        def outer(kk, carry):
            k0 = 4 * kk
            halfstep(k0, 0)
            halfstep(k0 + 1, 1)
            halfstep(k0 + 2, 2)
            halfstep(k0 + 3, 3)
            return carry

        lax.fori_loop(0, nch // 4, outer, 0)
        base = nch - nch % 4
        for j in range(nch % 4):
            halfstep(base + j, (base + j) % 4)
        pltpu.make_async_copy(ob, acc.at[idx0.at[1]], s_s).wait()
        plsc.subcore_barrier()
        pltpu.sync_copy(acc.at[pl.ds(zbase, RPT)],
                        out_hbm.at[ci, pl.ds(zbase, RPT)])

    kern = pl.kernel(
        body,
        out_type=jax.ShapeDtypeStruct((2, N_NODES, row), jnp.float32),
        mesh=mesh,
        scratch_types=[
            pltpu.VMEM((2, CHUNK), jnp.int32),
            pltpu.VMEM((2, CHUNK), jnp.int32),
            pltpu.VMEM((2, CHUNK), jnp.int32),
            pltpu.VMEM((2, CHUNK), jnp.int32),
            pltpu.VMEM((CHUNK, row), jnp.float32),
            pltpu.VMEM((CHUNK, row), jnp.float32),
            pltpu.VMEM((CHUNK, 16), jnp.float32),
            pltpu.VMEM((CHUNK, 16), jnp.float32),
            pltpu.VMEM((CHUNK, row), jnp.float32),
            pltpu.VMEM_SHARED((N_NODES, row), jnp.float32),
            pltpu.SemaphoreType.DMA,
            pltpu.SemaphoreType.DMA,
            pltpu.SemaphoreType.DMA,
            pltpu.SemaphoreType.DMA,
            pltpu.SemaphoreType.DMA,
            pltpu.SemaphoreType.DMA,
            pltpu.SemaphoreType.DMA,
            pltpu.SemaphoreType.DMA,
            pltpu.SemaphoreType.DMA,
        ],
        compiler_params=pltpu.CompilerParams(use_tc_tiling_on_sc=False),
    )
    return kern(table, er_table, eidx, zrows)


def kernel(features, edge_index, W1, attn_l1, attn_r1, b1,
           W2, attn_l2, attn_r2, res_W2, b2):
    eidx = edge_index.reshape(2, N_EDGES // CHUNK, CHUNK).transpose(1, 0, 2)
    al1 = attn_l1.reshape(1, 128)
    ar1 = attn_r1.reshape(1, 128)
    b1r = b1.reshape(1, 128)
    b2r = b2.reshape(1, 64)
    z144 = jnp.zeros((RPT, 144), jnp.float32)
    z80 = jnp.zeros((RPT, 80), jnp.float32)

    B = 1000
    grid = N_NODES // B

    t1, er1 = pl.pallas_call(
        _prep1_body,
        grid=(grid,),
        in_specs=[
            pl.BlockSpec((B, 128), lambda i: (i, 0)),
            pl.BlockSpec((128, 128), lambda i: (0, 0)),
            pl.BlockSpec((1, 128), lambda i: (0, 0)),
            pl.BlockSpec((1, 128), lambda i: (0, 0)),
        ],
        out_specs=[
            pl.BlockSpec((B, 144), lambda i: (i, 0)),
            pl.BlockSpec((B, 16), lambda i: (i, 0)),
        ],
        out_shape=[
            jax.ShapeDtypeStruct((N_NODES, 144), jnp.float32),
            jax.ShapeDtypeStruct((N_NODES, 16), jnp.float32),
        ],
    )(features, W1, al1, ar1)

    acc1 = _edge_pass(t1, er1, eidx, z144, row=144, nh=8)

    h, t2, er2 = pl.pallas_call(
        _mid_body,
        grid=(grid,),
        in_specs=[
            pl.BlockSpec((2, B, 144), lambda i: (0, i, 0)),
            pl.BlockSpec((B, 128), lambda i: (i, 0)),
            pl.BlockSpec((1, 128), lambda i: (0, 0)),
            pl.BlockSpec((128, 64), lambda i: (0, 0)),
            pl.BlockSpec((1, 64), lambda i: (0, 0)),
            pl.BlockSpec((1, 64), lambda i: (0, 0)),
        ],
        out_specs=[
            pl.BlockSpec((B, 128), lambda i: (i, 0)),
            pl.BlockSpec((B, 80), lambda i: (i, 0)),
            pl.BlockSpec((B, 16), lambda i: (i, 0)),
        ],
        out_shape=[
            jax.ShapeDtypeStruct((N_NODES, 128), jnp.float32),
            jax.ShapeDtypeStruct((N_NODES, 80), jnp.float32),
            jax.ShapeDtypeStruct((N_NODES, 16), jnp.float32),
        ],
    )(acc1, features, b1r, W2, attn_l2, attn_r2)

    acc2 = _edge_pass(t2, er2, eidx, z80, row=80, nh=1)

    out = pl.pallas_call(
        _fin_body,
        grid=(grid,),
        in_specs=[
            pl.BlockSpec((2, B, 80), lambda i: (0, i, 0)),
            pl.BlockSpec((B, 128), lambda i: (i, 0)),
            pl.BlockSpec((128, 64), lambda i: (0, 0)),
            pl.BlockSpec((1, 64), lambda i: (0, 0)),
        ],
        out_specs=pl.BlockSpec((B, 64), lambda i: (i, 0)),
        out_shape=jax.ShapeDtypeStruct((N_NODES, 64), jnp.float32),
    )(acc2, h, res_W2, b2r)

    return out


# speedup vs baseline: 1.0001x; 1.0001x over previous
import functools

import jax
import jax.numpy as jnp
from jax import lax
from jax.experimental import pallas as pl
from jax.experimental.pallas import tpu as pltpu
from jax.experimental.pallas import tpu_sc as plsc

N_NODES = 10000
N_EDGES = 320000
N_TILES = 32
EPT = N_EDGES // N_TILES
CHUNK = 80
RPT = N_NODES // 16


def _prep1_body(x_ref, w1_ref, al_ref, ar_ref, t1_ref, er1_ref):
    feat = jnp.dot(x_ref[...], w1_ref[...], preferred_element_type=jnp.float32)
    r = lax.broadcasted_iota(jnp.int32, (128, 8), 0) // 16
    c = lax.broadcasted_iota(jnp.int32, (128, 8), 1)
    sel = (r == c).astype(jnp.float32)
    el = jnp.dot(feat * al_ref[...], sel, preferred_element_type=jnp.float32)
    er = jnp.dot(feat * ar_ref[...], sel, preferred_element_type=jnp.float32)
    z8 = jnp.zeros_like(el)
    t1_ref[...] = jnp.concatenate([el, z8, feat], axis=1)
    er1_ref[...] = jnp.concatenate([er, z8], axis=1)


def _mid_body(acc_ref, x_ref, b1_ref, w2_ref, al2_ref, ar2_ref,
              h_ref, t2_ref, er2_ref):
    rows = acc_ref[0] + acc_ref[1]
    denom = rows[:, 0:8]
    msg = rows[:, 16:144]
    r = lax.broadcasted_iota(jnp.int32, (8, 128), 1) // 16
    c = lax.broadcasted_iota(jnp.int32, (8, 128), 0)
    sel = (r == c).astype(jnp.float32)
    den_e = jnp.maximum(
        jnp.dot(denom, sel, preferred_element_type=jnp.float32), 1e-9)
    rst = msg / den_e + x_ref[...] + b1_ref[...]
    h = jnp.where(rst > 0, rst, jnp.exp(rst) - 1.0)
    h_ref[...] = h
    feat2 = jnp.dot(h, w2_ref[...], preferred_element_type=jnp.float32)
    el2 = jnp.sum(feat2 * al2_ref[...], axis=1, keepdims=True)
    er2 = jnp.sum(feat2 * ar2_ref[...], axis=1, keepdims=True)
    z15 = jnp.zeros((el2.shape[0], 15), jnp.float32)
    t2_ref[...] = jnp.concatenate([el2, z15, feat2], axis=1)
    er2_ref[...] = jnp.concatenate([er2, z15], axis=1)


def _fin_body(acc_ref, h_ref, rw_ref, b2_ref, out_ref):
    rows = acc_ref[0] + acc_ref[1]
    denom = jnp.maximum(rows[:, 0:1], 1e-9)
    msg = rows[:, 16:80]
    res = jnp.dot(h_ref[...], rw_ref[...], preferred_element_type=jnp.float32)
    out_ref[...] = msg / denom + res + b2_ref[...]


def _edge_pass(table, er_table, eidx, zrows, *, row, nh):
    feat_regs = (row - 16) // 16
    nch = EPT // CHUNK
    mesh = plsc.VectorSubcoreMesh(core_axis_name="c", subcore_axis_name="s")

    def body(t_hbm, er_hbm, eidx_hbm, z_hbm, out_hbm,
             idx0, idx1, idx2, idx3, tr_a, tr_b, er_a, er_b, ob, acc,
             s_ta, s_tb, s_ea, s_eb, s_i0, s_i1, s_i2, s_i3, s_s):
        ci = lax.axis_index("c")
        si = lax.axis_index("s")
        tile = ci * 16 + si
        zbase = si * RPT
        rb = tile * nch
        pltpu.sync_copy(z_hbm, acc.at[pl.ds(zbase, RPT)])
        plsc.subcore_barrier()

        idxs = ((idx0, s_i0), (idx1, s_i1), (idx2, s_i2), (idx3, s_i3))
        trs = ((tr_a, s_ta, er_a, s_ea), (tr_b, s_tb, er_b, s_eb))

        def start_gather(j):
            tr, s_t, err, s_e = trs[j % 2]
            idx = idxs[j % 4][0]
            pltpu.async_copy(t_hbm.at[idx.at[0]], tr, s_t)
            pltpu.async_copy(er_hbm.at[idx.at[1]], err, s_e)

        def compute(tr, err):
            @plsc.parallel_loop(0, CHUNK, unroll=4)
            def edge(i):
                el = tr[i, pl.ds(0, 16)]
                er = err[i, pl.ds(0, 16)]
                e = el + er
                e = jnp.maximum(e, 0.2 * e)
                ex = jnp.exp(e)
                ob[i, pl.ds(0, 16)] = ex
                for j in range(feat_regs):
                    hh = j if nh > 1 else 0
                    a = ex.at[jnp.full((16,), hh, jnp.int32)].get(
                        mode="promise_in_bounds")
                    f = tr[i, pl.ds(16 + 16 * j, 16)]
                    ob[i, pl.ds(16 + 16 * j, 16)] = f * a

        def halfstep(k, j):
            tr, s_t, err, s_e = trs[j % 2]
            idx = idxs[j % 4][0]

            @pl.when(k + 1 < nch)
            def _():
                nidx, n_si = idxs[(j + 1) % 4]
                pltpu.make_async_copy(eidx_hbm.at[0], nidx, n_si).wait()
                start_gather(j + 1)

            @pl.when(k + 2 < nch)
            def _():
                nidx2, n_si2 = idxs[(j + 2) % 4]
                pltpu.async_copy(eidx_hbm.at[rb + k + 2], nidx2, n_si2)

            pltpu.make_async_copy(t_hbm.at[idx.at[0]], tr, s_t).wait()
            pltpu.make_async_copy(er_hbm.at[idx.at[1]], err, s_e).wait()

            @pl.when(k >= 1)
            def _():
                pltpu.make_async_copy(ob, acc.at[idx.at[1]], s_s).wait()

            compute(tr, err)
            pltpu.async_copy(ob, acc.at[idx.at[1]], s_s, add=True)

        pltpu.async_copy(eidx_hbm.at[rb], idx0, s_i0)
        pltpu.async_copy(eidx_hbm.at[rb + 1], idx1, s_i1)
        pltpu.make_async_copy(eidx_hbm.at[0], idx0, s_i0).wait()
        start_gather(0)

        def outer(kk, carry):
            k0 = 4 * kk
            halfstep(k0, 0)
            halfstep(k0 + 1, 1)
            halfstep(k0 + 2, 2)
            halfstep(k0 + 3, 3)
            return carry

        lax.fori_loop(0, nch // 4, outer, 0)
        base = nch - nch % 4
        for j in range(nch % 4):
            halfstep(base + j, (base + j) % 4)
        pltpu.make_async_copy(ob, acc.at[idx0.at[1]], s_s).wait()
        plsc.subcore_barrier()
        pltpu.sync_copy(acc.at[pl.ds(zbase, RPT)],
                        out_hbm.at[ci, pl.ds(zbase, RPT)])

    kern = pl.kernel(
        body,
        out_type=jax.ShapeDtypeStruct((2, N_NODES, row), jnp.float32),
        mesh=mesh,
        scratch_types=[
            pltpu.VMEM((2, CHUNK), jnp.int32),
            pltpu.VMEM((2, CHUNK), jnp.int32),
            pltpu.VMEM((2, CHUNK), jnp.int32),
            pltpu.VMEM((2, CHUNK), jnp.int32),
            pltpu.VMEM((CHUNK, row), jnp.float32),
            pltpu.VMEM((CHUNK, row), jnp.float32),
            pltpu.VMEM((CHUNK, 16), jnp.float32),
            pltpu.VMEM((CHUNK, 16), jnp.float32),
            pltpu.VMEM((CHUNK, row), jnp.float32),
            pltpu.VMEM_SHARED((N_NODES, row), jnp.float32),
            pltpu.SemaphoreType.DMA,
            pltpu.SemaphoreType.DMA,
            pltpu.SemaphoreType.DMA,
            pltpu.SemaphoreType.DMA,
            pltpu.SemaphoreType.DMA,
            pltpu.SemaphoreType.DMA,
            pltpu.SemaphoreType.DMA,
            pltpu.SemaphoreType.DMA,
            pltpu.SemaphoreType.DMA,
        ],
        compiler_params=pltpu.CompilerParams(use_tc_tiling_on_sc=False),
    )
    return kern(table, er_table, eidx, zrows)


def kernel(features, edge_index, W1, attn_l1, attn_r1, b1,
           W2, attn_l2, attn_r2, res_W2, b2):
    eidx = edge_index.reshape(2, N_EDGES // CHUNK, CHUNK).transpose(1, 0, 2)
    al1 = attn_l1.reshape(1, 128)
    ar1 = attn_r1.reshape(1, 128)
    b1r = b1.reshape(1, 128)
    b2r = b2.reshape(1, 64)
    z144 = jnp.zeros((RPT, 144), jnp.float32)
    z80 = jnp.zeros((RPT, 80), jnp.float32)

    B = 1000
    grid = N_NODES // B

    t1, er1 = pl.pallas_call(
        _prep1_body,
        grid=(grid,),
        in_specs=[
            pl.BlockSpec((B, 128), lambda i: (i, 0)),
            pl.BlockSpec((128, 128), lambda i: (0, 0)),
            pl.BlockSpec((1, 128), lambda i: (0, 0)),
            pl.BlockSpec((1, 128), lambda i: (0, 0)),
        ],
        out_specs=[
            pl.BlockSpec((B, 144), lambda i: (i, 0)),
            pl.BlockSpec((B, 16), lambda i: (i, 0)),
        ],
        out_shape=[
            jax.ShapeDtypeStruct((N_NODES, 144), jnp.float32),
            jax.ShapeDtypeStruct((N_NODES, 16), jnp.float32),
        ],
    )(features, W1, al1, ar1)

    acc1 = _edge_pass(t1, er1, eidx, z144, row=144, nh=8)

    h, t2, er2 = pl.pallas_call(
        _mid_body,
        grid=(grid,),
        in_specs=[
            pl.BlockSpec((2, B, 144), lambda i: (0, i, 0)),
            pl.BlockSpec((B, 128), lambda i: (i, 0)),
            pl.BlockSpec((1, 128), lambda i: (0, 0)),
            pl.BlockSpec((128, 64), lambda i: (0, 0)),
            pl.BlockSpec((1, 64), lambda i: (0, 0)),
            pl.BlockSpec((1, 64), lambda i: (0, 0)),
        ],
        out_specs=[
            pl.BlockSpec((B, 128), lambda i: (i, 0)),
            pl.BlockSpec((B, 80), lambda i: (i, 0)),
            pl.BlockSpec((B, 16), lambda i: (i, 0)),
        ],
        out_shape=[
            jax.ShapeDtypeStruct((N_NODES, 128), jnp.float32),
            jax.ShapeDtypeStruct((N_NODES, 80), jnp.float32),
            jax.ShapeDtypeStruct((N_NODES, 16), jnp.float32),
        ],
    )(acc1, features, b1r, W2, attn_l2, attn_r2)

    acc2 = _edge_pass(t2, er2, eidx, z80, row=80, nh=1)

    out = pl.pallas_call(
        _fin_body,
        grid=(grid,),
        in_specs=[
            pl.BlockSpec((2, B, 80), lambda i: (0, i, 0)),
            pl.BlockSpec((B, 128), lambda i: (i, 0)),
            pl.BlockSpec((128, 64), lambda i: (0, 0)),
            pl.BlockSpec((1, 64), lambda i: (0, 0)),
        ],
        out_specs=pl.BlockSpec((B, 64), lambda i: (i, 0)),
        out_shape=jax.ShapeDtypeStruct((N_NODES, 64), jnp.float32),
    )(acc2, h, res_W2, b2r)

    return out


# speedup vs baseline: 1.0178x; 1.0177x over previous
import functools

import jax
import jax.numpy as jnp
from jax import lax
from jax.experimental import pallas as pl
from jax.experimental.pallas import tpu as pltpu
from jax.experimental.pallas import tpu_sc as plsc

N_NODES = 10000
N_EDGES = 320000
N_TILES = 32
EPT = N_EDGES // N_TILES
CHUNK = 80
RPT = N_NODES // 16


def _prep1_body(x_ref, w1_ref, al_ref, ar_ref, t1_ref, er1_ref):
    feat = jnp.dot(x_ref[...], w1_ref[...], preferred_element_type=jnp.float32)
    r = lax.broadcasted_iota(jnp.int32, (128, 8), 0) // 16
    c = lax.broadcasted_iota(jnp.int32, (128, 8), 1)
    sel = (r == c).astype(jnp.float32)
    el = jnp.dot(feat * al_ref[...], sel, preferred_element_type=jnp.float32)
    er = jnp.dot(feat * ar_ref[...], sel, preferred_element_type=jnp.float32)
    z8 = jnp.zeros_like(el)
    t1_ref[...] = jnp.concatenate([el, z8, feat], axis=1)
    er1_ref[...] = jnp.concatenate([er, z8], axis=1)


def _mid_body(acc_ref, x_ref, b1_ref, w2_ref, al2_ref, ar2_ref,
              h_ref, t2_ref, er2_ref):
    rows = acc_ref[0] + acc_ref[1]
    denom = rows[:, 0:8]
    msg = rows[:, 16:144]
    r = lax.broadcasted_iota(jnp.int32, (8, 128), 1) // 16
    c = lax.broadcasted_iota(jnp.int32, (8, 128), 0)
    sel = (r == c).astype(jnp.float32)
    den_e = jnp.maximum(
        jnp.dot(denom, sel, preferred_element_type=jnp.float32), 1e-9)
    rst = msg / den_e + x_ref[...] + b1_ref[...]
    h = jnp.where(rst > 0, rst, jnp.exp(rst) - 1.0)
    h_ref[...] = h
    feat2 = jnp.dot(h, w2_ref[...], preferred_element_type=jnp.float32)
    el2 = jnp.sum(feat2 * al2_ref[...], axis=1, keepdims=True)
    er2 = jnp.sum(feat2 * ar2_ref[...], axis=1, keepdims=True)
    z15 = jnp.zeros((el2.shape[0], 15), jnp.float32)
    t2_ref[...] = jnp.concatenate([el2, z15, feat2], axis=1)
    er2_ref[...] = jnp.concatenate([er2, z15], axis=1)


def _fin_body(acc_ref, h_ref, rw_ref, b2_ref, out_ref):
    rows = acc_ref[0] + acc_ref[1]
    denom = jnp.maximum(rows[:, 0:1], 1e-9)
    msg = rows[:, 16:80]
    res = jnp.dot(h_ref[...], rw_ref[...], preferred_element_type=jnp.float32)
    out_ref[...] = msg / denom + res + b2_ref[...]


def _edge_pass(table, er_table, eidx, zrows, *, row, nh):
    feat_regs = (row - 16) // 16
    nch = EPT // CHUNK
    mesh = plsc.VectorSubcoreMesh(core_axis_name="c", subcore_axis_name="s")

    def body(t_hbm, er_hbm, eidx_hbm, z_hbm, out_hbm,
             idx0, idx1, idx2, idx3, tr_a, tr_b, er_a, er_b, ob, acc,
             s_ta, s_tb, s_ea, s_eb, s_i0, s_i1, s_i2, s_i3, s_s):
        ci = lax.axis_index("c")
        si = lax.axis_index("s")
        tile = ci * 16 + si
        zbase = si * RPT
        rb = tile * nch
        pltpu.sync_copy(z_hbm, acc.at[pl.ds(zbase, RPT)])
        plsc.subcore_barrier()

        idxs = ((idx0, s_i0), (idx1, s_i1), (idx2, s_i2), (idx3, s_i3))
        trs = ((tr_a, s_ta, er_a, s_ea), (tr_b, s_tb, er_b, s_eb))

        def start_gather(j):
            tr, s_t, err, s_e = trs[j % 2]
            idx = idxs[j % 4][0]
            pltpu.async_copy(t_hbm.at[idx.at[0]], tr, s_t)
            pltpu.async_copy(er_hbm.at[idx.at[1]], err, s_e)

        def compute(tr, err):
            @plsc.parallel_loop(0, CHUNK, unroll=4)
            def edge(i):
                el = tr[i, pl.ds(0, 16)]
                er = err[i, pl.ds(0, 16)]
                e = el + er
                e = jnp.maximum(e, 0.2 * e)
                ex = jnp.exp(e)
                ob[i, pl.ds(0, 16)] = ex
                for j in range(feat_regs):
                    hh = j if nh > 1 else 0
                    a = ex.at[jnp.full((16,), hh, jnp.int32)].get(
                        mode="promise_in_bounds")
                    f = tr[i, pl.ds(16 + 16 * j, 16)]
                    ob[i, pl.ds(16 + 16 * j, 16)] = f * a

        def halfstep(k, j):
            tr, s_t, err, s_e = trs[j % 2]
            idx = idxs[j % 4][0]

            @pl.when(k + 1 < nch)
            def _():
                nidx, n_si = idxs[(j + 1) % 4]
                pltpu.make_async_copy(eidx_hbm.at[0], nidx, n_si).wait()
                start_gather(j + 1)

            @pl.when(k + 2 < nch)
            def _():
                nidx2, n_si2 = idxs[(j + 2) % 4]
                pltpu.async_copy(eidx_hbm.at[rb + k + 2], nidx2, n_si2)

            pltpu.make_async_copy(t_hbm.at[idx.at[0]], tr, s_t).wait()
            pltpu.make_async_copy(er_hbm.at[idx.at[1]], err, s_e).wait()

            @pl.when(k >= 1)
            def _():
                pltpu.make_async_copy(ob, acc.at[idx.at[1]], s_s).wait()

            compute(tr, err)
            pltpu.async_copy(ob, acc.at[idx.at[1]], s_s, add=True)

        pltpu.async_copy(eidx_hbm.at[rb], idx0, s_i0)
        pltpu.async_copy(eidx_hbm.at[rb + 1], idx1, s_i1)
        pltpu.make_async_copy(eidx_hbm.at[0], idx0, s_i0).wait()
        start_gather(0)

        def outer(kk, carry):
            k0 = 4 * kk
            halfstep(k0, 0)
            halfstep(k0 + 1, 1)
            halfstep(k0 + 2, 2)
            halfstep(k0 + 3, 3)
            return carry

        lax.fori_loop(0, nch // 4, outer, 0)
        base = nch - nch % 4
        for j in range(nch % 4):
            halfstep(base + j, (base + j) % 4)
        pltpu.make_async_copy(ob, acc.at[idx0.at[1]], s_s).wait()
        plsc.subcore_barrier()
        pltpu.sync_copy(acc.at[pl.ds(zbase, RPT)],
                        out_hbm.at[ci, pl.ds(zbase, RPT)])

    kern = pl.kernel(
        body,
        out_type=jax.ShapeDtypeStruct((2, N_NODES, row), jnp.float32),
        mesh=mesh,
        scratch_types=[
            pltpu.VMEM((2, CHUNK), jnp.int32),
            pltpu.VMEM((2, CHUNK), jnp.int32),
            pltpu.VMEM((2, CHUNK), jnp.int32),
            pltpu.VMEM((2, CHUNK), jnp.int32),
            pltpu.VMEM((CHUNK, row), jnp.float32),
            pltpu.VMEM((CHUNK, row), jnp.float32),
            pltpu.VMEM((CHUNK, 16), jnp.float32),
            pltpu.VMEM((CHUNK, 16), jnp.float32),
            pltpu.VMEM((CHUNK, row), jnp.float32),
            pltpu.VMEM_SHARED((N_NODES, row), jnp.float32),
            pltpu.SemaphoreType.DMA,
            pltpu.SemaphoreType.DMA,
            pltpu.SemaphoreType.DMA,
            pltpu.SemaphoreType.DMA,
            pltpu.SemaphoreType.DMA,
            pltpu.SemaphoreType.DMA,
            pltpu.SemaphoreType.DMA,
            pltpu.SemaphoreType.DMA,
            pltpu.SemaphoreType.DMA,
        ],
        compiler_params=pltpu.CompilerParams(use_tc_tiling_on_sc=False),
    )
    return kern(table, er_table, eidx, zrows)


def kernel(features, edge_index, W1, attn_l1, attn_r1, b1,
           W2, attn_l2, attn_r2, res_W2, b2):
    eidx = edge_index.reshape(2, N_EDGES // CHUNK, CHUNK).transpose(1, 0, 2)
    al1 = attn_l1.reshape(1, 128)
    ar1 = attn_r1.reshape(1, 128)
    b1r = b1.reshape(1, 128)
    b2r = b2.reshape(1, 64)
    z144 = jnp.zeros((RPT, 144), jnp.float32)
    z80 = jnp.zeros((RPT, 80), jnp.float32)

    B = 2000
    grid = N_NODES // B

    t1, er1 = pl.pallas_call(
        _prep1_body,
        grid=(grid,),
        in_specs=[
            pl.BlockSpec((B, 128), lambda i: (i, 0)),
            pl.BlockSpec((128, 128), lambda i: (0, 0)),
            pl.BlockSpec((1, 128), lambda i: (0, 0)),
            pl.BlockSpec((1, 128), lambda i: (0, 0)),
        ],
        out_specs=[
            pl.BlockSpec((B, 144), lambda i: (i, 0)),
            pl.BlockSpec((B, 16), lambda i: (i, 0)),
        ],
        out_shape=[
            jax.ShapeDtypeStruct((N_NODES, 144), jnp.float32),
            jax.ShapeDtypeStruct((N_NODES, 16), jnp.float32),
        ],
    )(features, W1, al1, ar1)

    acc1 = _edge_pass(t1, er1, eidx, z144, row=144, nh=8)

    h, t2, er2 = pl.pallas_call(
        _mid_body,
        grid=(grid,),
        in_specs=[
            pl.BlockSpec((2, B, 144), lambda i: (0, i, 0)),
            pl.BlockSpec((B, 128), lambda i: (i, 0)),
            pl.BlockSpec((1, 128), lambda i: (0, 0)),
            pl.BlockSpec((128, 64), lambda i: (0, 0)),
            pl.BlockSpec((1, 64), lambda i: (0, 0)),
            pl.BlockSpec((1, 64), lambda i: (0, 0)),
        ],
        out_specs=[
            pl.BlockSpec((B, 128), lambda i: (i, 0)),
            pl.BlockSpec((B, 80), lambda i: (i, 0)),
            pl.BlockSpec((B, 16), lambda i: (i, 0)),
        ],
        out_shape=[
            jax.ShapeDtypeStruct((N_NODES, 128), jnp.float32),
            jax.ShapeDtypeStruct((N_NODES, 80), jnp.float32),
            jax.ShapeDtypeStruct((N_NODES, 16), jnp.float32),
        ],
    )(acc1, features, b1r, W2, attn_l2, attn_r2)

    acc2 = _edge_pass(t2, er2, eidx, z80, row=80, nh=1)

    out = pl.pallas_call(
        _fin_body,
        grid=(grid,),
        in_specs=[
            pl.BlockSpec((2, B, 80), lambda i: (0, i, 0)),
            pl.BlockSpec((B, 128), lambda i: (i, 0)),
            pl.BlockSpec((128, 64), lambda i: (0, 0)),
            pl.BlockSpec((1, 64), lambda i: (0, 0)),
        ],
        out_specs=pl.BlockSpec((B, 64), lambda i: (i, 0)),
        out_shape=jax.ShapeDtypeStruct((N_NODES, 64), jnp.float32),
    )(acc2, h, res_W2, b2r)

    return out


# speedup vs baseline: 1.0244x; 1.0064x over previous
import functools

import jax
import jax.numpy as jnp
from jax import lax
from jax.experimental import pallas as pl
from jax.experimental.pallas import tpu as pltpu
from jax.experimental.pallas import tpu_sc as plsc

N_NODES = 10000
N_EDGES = 320000
N_TILES = 32
EPT = N_EDGES // N_TILES
CHUNK = 80
RPT = N_NODES // 16


def _prep1_body(x_ref, w1_ref, al_ref, ar_ref, t1_ref, er1_ref):
    feat = jnp.dot(x_ref[...], w1_ref[...], preferred_element_type=jnp.float32)
    r = lax.broadcasted_iota(jnp.int32, (128, 8), 0) // 16
    c = lax.broadcasted_iota(jnp.int32, (128, 8), 1)
    sel = (r == c).astype(jnp.float32)
    el = jnp.dot(feat * al_ref[...], sel, preferred_element_type=jnp.float32)
    er = jnp.dot(feat * ar_ref[...], sel, preferred_element_type=jnp.float32)
    z8 = jnp.zeros_like(el)
    t1_ref[...] = jnp.concatenate([el, z8, feat], axis=1)
    er1_ref[...] = jnp.concatenate([er, z8], axis=1)


def _mid_body(acc_ref, x_ref, b1_ref, w2_ref, al2_ref, ar2_ref,
              h_ref, t2_ref, er2_ref):
    rows = acc_ref[0] + acc_ref[1]
    denom = rows[:, 0:8]
    msg = rows[:, 16:144]
    r = lax.broadcasted_iota(jnp.int32, (8, 128), 1) // 16
    c = lax.broadcasted_iota(jnp.int32, (8, 128), 0)
    sel = (r == c).astype(jnp.float32)
    den_e = jnp.maximum(
        jnp.dot(denom, sel, preferred_element_type=jnp.float32), 1e-9)
    rst = msg / den_e + x_ref[...] + b1_ref[...]
    h = jnp.where(rst > 0, rst, jnp.exp(rst) - 1.0)
    h_ref[...] = h
    feat2 = jnp.dot(h, w2_ref[...], preferred_element_type=jnp.float32)
    el2 = jnp.sum(feat2 * al2_ref[...], axis=1, keepdims=True)
    er2 = jnp.sum(feat2 * ar2_ref[...], axis=1, keepdims=True)
    z15 = jnp.zeros((el2.shape[0], 15), jnp.float32)
    t2_ref[...] = jnp.concatenate([el2, z15, feat2], axis=1)
    er2_ref[...] = jnp.concatenate([er2, z15], axis=1)


def _fin_body(acc_ref, h_ref, rw_ref, b2_ref, out_ref):
    rows = acc_ref[0] + acc_ref[1]
    denom = jnp.maximum(rows[:, 0:1], 1e-9)
    msg = rows[:, 16:80]
    res = jnp.dot(h_ref[...], rw_ref[...], preferred_element_type=jnp.float32)
    out_ref[...] = msg / denom + res + b2_ref[...]


def _edge_pass(table, er_table, eidx, zrows, *, row, nh):
    feat_regs = (row - 16) // 16
    nch = EPT // CHUNK
    mesh = plsc.VectorSubcoreMesh(core_axis_name="c", subcore_axis_name="s")

    def body(t_hbm, er_hbm, eidx_hbm, z_hbm, out_hbm,
             idx0, idx1, idx2, idx3, tr_a, tr_b, er_a, er_b, ob, acc,
             s_ta, s_tb, s_ea, s_eb, s_i0, s_i1, s_i2, s_i3, s_s):
        ci = lax.axis_index("c")
        si = lax.axis_index("s")
        tile = ci * 16 + si
        zbase = si * RPT
        rb = tile * nch
        pltpu.sync_copy(z_hbm, acc.at[pl.ds(zbase, RPT)])
        plsc.subcore_barrier()

        idxs = ((idx0, s_i0), (idx1, s_i1), (idx2, s_i2), (idx3, s_i3))
        trs = ((tr_a, s_ta, er_a, s_ea), (tr_b, s_tb, er_b, s_eb))

        def start_gather(j):
            tr, s_t, err, s_e = trs[j % 2]
            idx = idxs[j % 4][0]
            pltpu.async_copy(t_hbm.at[idx.at[0]], tr, s_t)
            pltpu.async_copy(er_hbm.at[idx.at[1]], err, s_e)

        def compute(tr, err):
            @plsc.parallel_loop(0, CHUNK, unroll=4)
            def edge(i):
                el = tr[i, pl.ds(0, 16)]
                er = err[i, pl.ds(0, 16)]
                e = el + er
                e = jnp.maximum(e, 0.2 * e)
                ex = jnp.exp(e)
                ob[i, pl.ds(0, 16)] = ex
                for j in range(feat_regs):
                    hh = j if nh > 1 else 0
                    a = ex.at[jnp.full((16,), hh, jnp.int32)].get(
                        mode="promise_in_bounds")
                    f = tr[i, pl.ds(16 + 16 * j, 16)]
                    ob[i, pl.ds(16 + 16 * j, 16)] = f * a

        def halfstep(k, j):
            tr, s_t, err, s_e = trs[j % 2]
            idx = idxs[j % 4][0]

            @pl.when(k + 1 < nch)
            def _():
                nidx, n_si = idxs[(j + 1) % 4]
                pltpu.make_async_copy(eidx_hbm.at[0], nidx, n_si).wait()
                start_gather(j + 1)

            @pl.when(k + 2 < nch)
            def _():
                nidx2, n_si2 = idxs[(j + 2) % 4]
                pltpu.async_copy(eidx_hbm.at[rb + k + 2], nidx2, n_si2)

            pltpu.make_async_copy(t_hbm.at[idx.at[0]], tr, s_t).wait()
            pltpu.make_async_copy(er_hbm.at[idx.at[1]], err, s_e).wait()

            @pl.when(k >= 1)
            def _():
                pltpu.make_async_copy(ob, acc.at[idx.at[1]], s_s).wait()

            compute(tr, err)
            pltpu.async_copy(ob, acc.at[idx.at[1]], s_s, add=True)

        pltpu.async_copy(eidx_hbm.at[rb], idx0, s_i0)
        pltpu.async_copy(eidx_hbm.at[rb + 1], idx1, s_i1)
        pltpu.make_async_copy(eidx_hbm.at[0], idx0, s_i0).wait()
        start_gather(0)

        def outer(kk, carry):
            k0 = 4 * kk
            halfstep(k0, 0)
            halfstep(k0 + 1, 1)
            halfstep(k0 + 2, 2)
            halfstep(k0 + 3, 3)
            return carry

        lax.fori_loop(0, nch // 4, outer, 0)
        base = nch - nch % 4
        for j in range(nch % 4):
            halfstep(base + j, (base + j) % 4)
        pltpu.make_async_copy(ob, acc.at[idx0.at[1]], s_s).wait()
        plsc.subcore_barrier()
        pltpu.sync_copy(acc.at[pl.ds(zbase, RPT)],
                        out_hbm.at[ci, pl.ds(zbase, RPT)])

    kern = pl.kernel(
        body,
        out_type=jax.ShapeDtypeStruct((2, N_NODES, row), jnp.float32),
        mesh=mesh,
        scratch_types=[
            pltpu.VMEM((2, CHUNK), jnp.int32),
            pltpu.VMEM((2, CHUNK), jnp.int32),
            pltpu.VMEM((2, CHUNK), jnp.int32),
            pltpu.VMEM((2, CHUNK), jnp.int32),
            pltpu.VMEM((CHUNK, row), jnp.float32),
            pltpu.VMEM((CHUNK, row), jnp.float32),
            pltpu.VMEM((CHUNK, 16), jnp.float32),
            pltpu.VMEM((CHUNK, 16), jnp.float32),
            pltpu.VMEM((CHUNK, row), jnp.float32),
            pltpu.VMEM_SHARED((N_NODES, row), jnp.float32),
            pltpu.SemaphoreType.DMA,
            pltpu.SemaphoreType.DMA,
            pltpu.SemaphoreType.DMA,
            pltpu.SemaphoreType.DMA,
            pltpu.SemaphoreType.DMA,
            pltpu.SemaphoreType.DMA,
            pltpu.SemaphoreType.DMA,
            pltpu.SemaphoreType.DMA,
            pltpu.SemaphoreType.DMA,
        ],
        compiler_params=pltpu.CompilerParams(use_tc_tiling_on_sc=False),
    )
    return kern(table, er_table, eidx, zrows)


def kernel(features, edge_index, W1, attn_l1, attn_r1, b1,
           W2, attn_l2, attn_r2, res_W2, b2):
    eidx = edge_index.reshape(2, N_EDGES // CHUNK, CHUNK).transpose(1, 0, 2)
    al1 = attn_l1.reshape(1, 128)
    ar1 = attn_r1.reshape(1, 128)
    b1r = b1.reshape(1, 128)
    b2r = b2.reshape(1, 64)
    z144 = jnp.zeros((RPT, 144), jnp.float32)
    z80 = jnp.zeros((RPT, 80), jnp.float32)

    B = 5000
    grid = N_NODES // B

    t1, er1 = pl.pallas_call(
        _prep1_body,
        grid=(grid,),
        in_specs=[
            pl.BlockSpec((B, 128), lambda i: (i, 0)),
            pl.BlockSpec((128, 128), lambda i: (0, 0)),
            pl.BlockSpec((1, 128), lambda i: (0, 0)),
            pl.BlockSpec((1, 128), lambda i: (0, 0)),
        ],
        out_specs=[
            pl.BlockSpec((B, 144), lambda i: (i, 0)),
            pl.BlockSpec((B, 16), lambda i: (i, 0)),
        ],
        out_shape=[
            jax.ShapeDtypeStruct((N_NODES, 144), jnp.float32),
            jax.ShapeDtypeStruct((N_NODES, 16), jnp.float32),
        ],
    )(features, W1, al1, ar1)

    acc1 = _edge_pass(t1, er1, eidx, z144, row=144, nh=8)

    h, t2, er2 = pl.pallas_call(
        _mid_body,
        grid=(grid,),
        in_specs=[
            pl.BlockSpec((2, B, 144), lambda i: (0, i, 0)),
            pl.BlockSpec((B, 128), lambda i: (i, 0)),
            pl.BlockSpec((1, 128), lambda i: (0, 0)),
            pl.BlockSpec((128, 64), lambda i: (0, 0)),
            pl.BlockSpec((1, 64), lambda i: (0, 0)),
            pl.BlockSpec((1, 64), lambda i: (0, 0)),
        ],
        out_specs=[
            pl.BlockSpec((B, 128), lambda i: (i, 0)),
            pl.BlockSpec((B, 80), lambda i: (i, 0)),
            pl.BlockSpec((B, 16), lambda i: (i, 0)),
        ],
        out_shape=[
            jax.ShapeDtypeStruct((N_NODES, 128), jnp.float32),
            jax.ShapeDtypeStruct((N_NODES, 80), jnp.float32),
            jax.ShapeDtypeStruct((N_NODES, 16), jnp.float32),
        ],
    )(acc1, features, b1r, W2, attn_l2, attn_r2)

    acc2 = _edge_pass(t2, er2, eidx, z80, row=80, nh=1)

    out = pl.pallas_call(
        _fin_body,
        grid=(grid,),
        in_specs=[
            pl.BlockSpec((2, B, 80), lambda i: (0, i, 0)),
            pl.BlockSpec((B, 128), lambda i: (i, 0)),
            pl.BlockSpec((128, 64), lambda i: (0, 0)),
            pl.BlockSpec((1, 64), lambda i: (0, 0)),
        ],
        out_specs=pl.BlockSpec((B, 64), lambda i: (i, 0)),
        out_shape=jax.ShapeDtypeStruct((N_NODES, 64), jnp.float32),
    )(acc2, h, res_W2, b2r)

    return out
